# baseline (speedup 1.0000x reference)
import jax
import jax.numpy as jnp
from jax import lax
from jax.experimental import pallas as pl
from jax.experimental.pallas import tpu as pltpu
from jax.experimental.pallas import tpu_sc as plsc

B, L = 4096, 200
OUT_D = 96

_SRC = [
    ("bio", 3, 16),
    ("pos", 19, 16),
    ("ner", 19, 16),
    ("ans", 3, 8),
    ("clue", 2, 8),
    ("acr", 2, 8),
    ("acap", 2, 8),
    ("cap", 2, 8),
    ("pnum", 11, 8),
]

_TBL_BASE = []
_off = 0
for _n, _v, _d in _SRC:
    _TBL_BASE.append(_off)
    _off += -(-(_v * _d) // 8) * 8
_TBL_WORDS = _off

T0_ROWS = 3 * 19 * 19
T1_ROWS = 3 * 2 * 2 * 2 * 2 * 11

NW = 32
ROWS_PER_W = B // NW
GROUPS_PAD = (L + 15) // 16
L_PAD = GROUPS_PAD * 16

T0_PER_S = -(-T0_ROWS // 16)
T1_PER_S = -(-T1_ROWS // 16)


def _body(*refs):
    tag_hbm = refs[0]
    w_hbm = refs[1:10]
    out_hbm = refs[10]
    t0_sp = refs[11]
    t1_sp = refs[12]
    base_v = refs[13]
    stage_v = refs[14]
    tags_v = [refs[15:24], refs[24:33]]
    pidx_v = [refs[33:35], refs[35:37]]
    row_v = [refs[37:39], refs[39:41]]
    sem_t = refs[41:43]
    sem_g = refs[43:45]
    sem_w = refs[45:47]

    cid = lax.axis_index("c")
    sid = lax.axis_index("s")
    wid = sid * 2 + cid
    base_row = wid * ROWS_PER_W

    lane = lax.iota(jnp.int32, 16)

    for t in range(9):
        v, d = _SRC[t][1], _SRC[t][2]
        pltpu.sync_copy(w_hbm[t], base_v.at[pl.ds(_TBL_BASE[t], v * d)])

    def build_t0(i, carry):
        r = sid * T0_PER_S + i
        b = r // (19 * 19)
        p = (r // 19) % 19
        n = r % 19
        idx_b = _TBL_BASE[0] + b * 16 + lane
        idx_p = _TBL_BASE[1] + p * 16 + lane
        idx_n = _TBL_BASE[2] + n * 16 + lane
        stage_v[i, pl.ds(0, 16)] = plsc.load_gather(base_v, [idx_b])
        stage_v[i, pl.ds(16, 16)] = plsc.load_gather(base_v, [idx_p])
        stage_v[i, pl.ds(32, 16)] = plsc.load_gather(base_v, [idx_n])
        return carry

    n0 = jnp.minimum(T0_PER_S, jnp.maximum(0, T0_ROWS - sid * T0_PER_S))
    lax.fori_loop(0, n0, build_t0, 0, unroll=False)

    def flush0(i, carry):
        r = sid * T0_PER_S + i
        pltpu.sync_copy(stage_v.at[i], t0_sp.at[r])
        return carry
    lax.fori_loop(0, n0, flush0, 0, unroll=False)

    def build_t1(i, carry):
        r = sid * T1_PER_S + i
        a = r // (2 * 2 * 2 * 2 * 11)
        c = (r // (2 * 2 * 2 * 11)) % 2
        ar = (r // (2 * 2 * 11)) % 2
        aa = (r // (2 * 11)) % 2
        cp = (r // 11) % 2
        pn = r % 11
        lo = lane < 8
        l8 = lane - 8
        pairs = [(3, a, 4, c), (5, ar, 6, aa), (7, cp, 8, pn)]
        for g, (ta, ia, tb, ib) in enumerate(pairs):
            idx = jnp.where(lo, _TBL_BASE[ta] + ia * 8 + lane,
                            _TBL_BASE[tb] + ib * 8 + l8)
            stage_v[i, pl.ds(g * 16, 16)] = plsc.load_gather(base_v, [idx])
        return carry

    n1 = jnp.minimum(T1_PER_S, jnp.maximum(0, T1_ROWS - sid * T1_PER_S))
    lax.fori_loop(0, n1, build_t1, 0, unroll=False)

    def flush1(i, carry):
        r = sid * T1_PER_S + i
        pltpu.sync_copy(stage_v.at[i], t1_sp.at[r])
        return carry
    lax.fori_loop(0, n1, flush1, 0, unroll=False)

    plsc.subcore_barrier()

    zeros = jnp.zeros((16,), jnp.int32)
    for p in (0, 1):
        for t in range(9):
            tags_v[p][t][pl.ds(L_PAD - 16, 16)] = zeros

    idx_refs = [[pidx_v[p][0], pidx_v[p][1]] for p in (0, 1)]
    tbl_refs = [t0_sp, t1_sp]

    def issue_tags(p, k):
        off = (base_row + k) * L
        for t in range(9):
            pltpu.async_copy(tag_hbm.at[t, pl.ds(off, L)],
                             tags_v[p][t].at[pl.ds(0, L)], sem_t[p])

    def wait_tags(p):
        for t in range(9):
            pltpu.make_async_copy(tag_hbm.at[t, pl.ds(0, L)],
                                  tags_v[p][t].at[pl.ds(0, L)],
                                  sem_t[p]).wait()

    def compute_pidx(p):
        def group_body(g, carry):
            s = pl.ds(g * 16, 16)
            tv = [tags_v[p][t][s] for t in range(9)]
            pidx_v[p][0][s] = (tv[0] * 19 + tv[1]) * 19 + tv[2]
            pidx_v[p][1][s] = ((((tv[3] * 2 + tv[4]) * 2 + tv[5]) * 2
                               + tv[6]) * 2 + tv[7]) * 11 + tv[8]
            return carry
        lax.fori_loop(0, GROUPS_PAD, group_body, 0, unroll=False)

    def issue_gathers(p):
        for m in range(2):
            pltpu.async_copy(tbl_refs[m].at[idx_refs[p][m]], row_v[p][m],
                             sem_g[p])

    def wait_gathers(p):
        for m in range(2):
            pltpu.make_async_copy(tbl_refs[m].at[idx_refs[p][m]], row_v[p][m],
                                  sem_g[p]).wait()

    def issue_writes(p, k):
        row = base_row + k
        for m in range(2):
            pltpu.async_copy(row_v[p][m].at[pl.ds(0, L), :],
                             out_hbm.at[row, :, pl.ds(m * 48, 48)],
                             sem_w[p])

    def wait_writes(p):
        for m in range(2):
            pltpu.make_async_copy(row_v[p][m].at[pl.ds(0, L), :],
                                  out_hbm.at[0, :, pl.ds(m * 48, 48)],
                                  sem_w[p]).wait()

    issue_tags(0, 0)

    def pair_body(j, carry):
        for p in (0, 1):
            k = j * 2 + p
            wait_tags(p)
            compute_pidx(p)

            @pl.when(j >= 1)
            def _drain_writes():
                wait_writes(p)

            issue_gathers(p)

            @pl.when((j >= 1) | (p == 1))
            def _flush_prev():
                wait_gathers(1 - p)
                issue_writes(1 - p, k - 1)

            if p == 0:
                issue_tags(1, k + 1)
            else:
                @pl.when(j < ROWS_PER_W // 2 - 1)
                def _prefetch():
                    issue_tags(0, k + 1)
        return carry

    lax.fori_loop(0, ROWS_PER_W // 2, pair_body, 0, unroll=False)

    wait_gathers(1)
    issue_writes(1, ROWS_PER_W - 1)
    wait_writes(0)
    wait_writes(1)


@jax.jit
def kernel(bio_tag, ner_tag, pos_tag, ans_tag, clue_tag, acr_tag, acap_tag,
           cap_tag, pnum_tag, bio_w, ner_w, pos_w, ans_w, clue_w, acr_w,
           acap_w, cap_w, pnum_w):
    tags = {"bio": bio_tag, "ner": ner_tag, "pos": pos_tag, "ans": ans_tag,
            "clue": clue_tag, "acr": acr_tag, "acap": acap_tag, "cap": cap_tag,
            "pnum": pnum_tag}
    ws = {"bio": bio_w, "ner": ner_w, "pos": pos_w, "ans": ans_w,
          "clue": clue_w, "acr": acr_w, "acap": acap_w, "cap": cap_w,
          "pnum": pnum_w}
    tag_in = jnp.stack([tags[n].reshape(-1).astype(jnp.int32)
                        for n, _v, _d in _SRC])
    w_in = [ws[n].reshape(-1).astype(jnp.float32) for n, _v, _d in _SRC]

    mesh = plsc.VectorSubcoreMesh(core_axis_name="c", subcore_axis_name="s")
    run = pl.kernel(
        _body,
        out_type=jax.ShapeDtypeStruct((B, L, 128), jnp.float32),
        mesh=mesh,
        compiler_params=pltpu.CompilerParams(needs_layout_passes=False,
                                             use_tc_tiling_on_sc=False),
        scratch_types=[
            pltpu.VMEM_SHARED((T0_ROWS, 48), jnp.float32),
            pltpu.VMEM_SHARED((T1_ROWS, 48), jnp.float32),
            pltpu.VMEM((_TBL_WORDS,), jnp.float32),
            pltpu.VMEM((T0_PER_S, 48), jnp.float32),
            *[pltpu.VMEM((L_PAD,), jnp.int32) for _ in range(18)],
            *[pltpu.VMEM((L_PAD,), jnp.int32) for _ in range(4)],
            *[pltpu.VMEM((L_PAD, 48), jnp.float32) for _ in range(4)],
            *[pltpu.SemaphoreType.DMA for _ in range(6)],
        ],
    )
    return run(tag_in, *w_in)[:, :, :OUT_D]


# speedup vs baseline: 2.1671x; 2.1671x over previous
import jax
import jax.numpy as jnp
from jax import lax
from jax.experimental import pallas as pl
from jax.experimental.pallas import tpu as pltpu
from jax.experimental.pallas import tpu_sc as plsc

B, L = 4096, 200
OUT_D = 96

_SRC = [
    ("bio", 3, 16),
    ("pos", 19, 16),
    ("ner", 19, 16),
    ("ans", 3, 8),
    ("clue", 2, 8),
    ("acr", 2, 8),
    ("acap", 2, 8),
    ("cap", 2, 8),
    ("pnum", 11, 8),
]

_TBL_BASE = []
_off = 0
for _n, _v, _d in _SRC:
    _TBL_BASE.append(_off)
    _off += -(-(_v * _d) // 8) * 8
_TBL_WORDS = _off

T0_ROWS = 3 * 19 * 19
T1_ROWS = 3 * 2 * 2 * 2 * 2 * 11

NW = 32
ROWS_PER_W = B // NW
GROUPS_PAD = (L + 15) // 16
L_PAD = GROUPS_PAD * 16

T0_PER_S = -(-T0_ROWS // 16)
T1_PER_S = -(-T1_ROWS // 16)


def _body(*refs):
    tag_hbm = refs[0:9]
    w_hbm = refs[9:18]
    out_hbm = refs[18]
    t0_sp = refs[19]
    t1_sp = refs[20]
    base_v = refs[21]
    stage_v = refs[22]
    tags_v = [refs[23:32], refs[32:41]]
    pidx_v = [refs[41:43], refs[43:45]]
    row_v = [refs[45:47], refs[47:49]]
    sem_t = refs[49:51]
    sem_g = refs[51:53]
    sem_w = refs[53:55]

    cid = lax.axis_index("c")
    sid = lax.axis_index("s")
    wid = sid * 2 + cid
    base_row = wid * ROWS_PER_W

    lane = lax.iota(jnp.int32, 16)

    for t in range(9):
        v, d = _SRC[t][1], _SRC[t][2]
        pltpu.sync_copy(w_hbm[t], base_v.at[pl.ds(_TBL_BASE[t], v * d)])

    def build_t0(i, carry):
        r = sid * T0_PER_S + i
        b = r // (19 * 19)
        p = (r // 19) % 19
        n = r % 19
        idx_b = _TBL_BASE[0] + b * 16 + lane
        idx_p = _TBL_BASE[1] + p * 16 + lane
        idx_n = _TBL_BASE[2] + n * 16 + lane
        stage_v[i, pl.ds(0, 16)] = plsc.load_gather(base_v, [idx_b])
        stage_v[i, pl.ds(16, 16)] = plsc.load_gather(base_v, [idx_p])
        stage_v[i, pl.ds(32, 16)] = plsc.load_gather(base_v, [idx_n])
        return carry

    n0 = jnp.minimum(T0_PER_S, jnp.maximum(0, T0_ROWS - sid * T0_PER_S))
    lax.fori_loop(0, n0, build_t0, 0, unroll=False)

    def flush0(i, carry):
        r = sid * T0_PER_S + i
        pltpu.sync_copy(stage_v.at[i], t0_sp.at[r])
        return carry
    lax.fori_loop(0, n0, flush0, 0, unroll=False)

    def build_t1(i, carry):
        r = sid * T1_PER_S + i
        a = r // (2 * 2 * 2 * 2 * 11)
        c = (r // (2 * 2 * 2 * 11)) % 2
        ar = (r // (2 * 2 * 11)) % 2
        aa = (r // (2 * 11)) % 2
        cp = (r // 11) % 2
        pn = r % 11
        lo = lane < 8
        l8 = lane - 8
        pairs = [(3, a, 4, c), (5, ar, 6, aa), (7, cp, 8, pn)]
        for g, (ta, ia, tb, ib) in enumerate(pairs):
            idx = jnp.where(lo, _TBL_BASE[ta] + ia * 8 + lane,
                            _TBL_BASE[tb] + ib * 8 + l8)
            stage_v[i, pl.ds(g * 16, 16)] = plsc.load_gather(base_v, [idx])
        return carry

    n1 = jnp.minimum(T1_PER_S, jnp.maximum(0, T1_ROWS - sid * T1_PER_S))
    lax.fori_loop(0, n1, build_t1, 0, unroll=False)

    def flush1(i, carry):
        r = sid * T1_PER_S + i
        pltpu.sync_copy(stage_v.at[i], t1_sp.at[r])
        return carry
    lax.fori_loop(0, n1, flush1, 0, unroll=False)

    plsc.subcore_barrier()

    zeros = jnp.zeros((16,), jnp.int32)
    for p in (0, 1):
        for t in range(9):
            tags_v[p][t][pl.ds(L_PAD - 16, 16)] = zeros

    idx_refs = [[pidx_v[p][0], pidx_v[p][1]] for p in (0, 1)]
    tbl_refs = [t0_sp, t1_sp]

    def issue_tags(p, k):
        row = base_row + k
        for t in range(9):
            pltpu.async_copy(tag_hbm[t].at[row], tags_v[p][t].at[pl.ds(0, L)],
                             sem_t[p])

    def wait_tags(p):
        for t in range(9):
            pltpu.make_async_copy(tag_hbm[t].at[0],
                                  tags_v[p][t].at[pl.ds(0, L)],
                                  sem_t[p]).wait()

    def compute_pidx(p):
        def group_body(g, carry):
            s = pl.ds(g * 16, 16)
            tv = [tags_v[p][t][s] for t in range(9)]
            pidx_v[p][0][s] = (tv[0] * 19 + tv[1]) * 19 + tv[2]
            pidx_v[p][1][s] = ((((tv[3] * 2 + tv[4]) * 2 + tv[5]) * 2
                               + tv[6]) * 2 + tv[7]) * 11 + tv[8]
            return carry
        lax.fori_loop(0, GROUPS_PAD, group_body, 0, unroll=False)

    def issue_gathers(p):
        for m in range(2):
            pltpu.async_copy(tbl_refs[m].at[idx_refs[p][m]], row_v[p][m],
                             sem_g[p])

    def wait_gathers(p):
        for m in range(2):
            pltpu.make_async_copy(tbl_refs[m].at[idx_refs[p][m]], row_v[p][m],
                                  sem_g[p]).wait()

    def issue_writes(p, k):
        row = base_row + k
        for m in range(2):
            pltpu.async_copy(row_v[p][m].at[pl.ds(0, L), :],
                             out_hbm.at[row, :, pl.ds(m * 48, 48)],
                             sem_w[p])

    def wait_writes(p):
        for m in range(2):
            pltpu.make_async_copy(row_v[p][m].at[pl.ds(0, L), :],
                                  out_hbm.at[0, :, pl.ds(m * 48, 48)],
                                  sem_w[p]).wait()

    issue_tags(0, 0)

    def pair_body(j, carry):
        for p in (0, 1):
            k = j * 2 + p
            wait_tags(p)
            compute_pidx(p)

            @pl.when(j >= 1)
            def _drain_writes():
                wait_writes(p)

            issue_gathers(p)

            @pl.when((j >= 1) | (p == 1))
            def _flush_prev():
                wait_gathers(1 - p)
                issue_writes(1 - p, k - 1)

            if p == 0:
                issue_tags(1, k + 1)
            else:
                @pl.when(j < ROWS_PER_W // 2 - 1)
                def _prefetch():
                    issue_tags(0, k + 1)
        return carry

    lax.fori_loop(0, ROWS_PER_W // 2, pair_body, 0, unroll=False)

    wait_gathers(1)
    issue_writes(1, ROWS_PER_W - 1)
    wait_writes(0)
    wait_writes(1)


@jax.jit
def kernel(bio_tag, ner_tag, pos_tag, ans_tag, clue_tag, acr_tag, acap_tag,
           cap_tag, pnum_tag, bio_w, ner_w, pos_w, ans_w, clue_w, acr_w,
           acap_w, cap_w, pnum_w):
    tags = {"bio": bio_tag, "ner": ner_tag, "pos": pos_tag, "ans": ans_tag,
            "clue": clue_tag, "acr": acr_tag, "acap": acap_tag, "cap": cap_tag,
            "pnum": pnum_tag}
    ws = {"bio": bio_w, "ner": ner_w, "pos": pos_w, "ans": ans_w,
          "clue": clue_w, "acr": acr_w, "acap": acap_w, "cap": cap_w,
          "pnum": pnum_w}
    tag_in = [tags[n].astype(jnp.int32) for n, _v, _d in _SRC]
    w_in = [ws[n].reshape(-1).astype(jnp.float32) for n, _v, _d in _SRC]

    mesh = plsc.VectorSubcoreMesh(core_axis_name="c", subcore_axis_name="s")
    run = pl.kernel(
        _body,
        out_type=jax.ShapeDtypeStruct((B, L, 128), jnp.float32),
        mesh=mesh,
        compiler_params=pltpu.CompilerParams(needs_layout_passes=False,
                                             use_tc_tiling_on_sc=False),
        scratch_types=[
            pltpu.VMEM_SHARED((T0_ROWS, 48), jnp.float32),
            pltpu.VMEM_SHARED((T1_ROWS, 48), jnp.float32),
            pltpu.VMEM((_TBL_WORDS,), jnp.float32),
            pltpu.VMEM((T0_PER_S, 48), jnp.float32),
            *[pltpu.VMEM((L_PAD,), jnp.int32) for _ in range(18)],
            *[pltpu.VMEM((L_PAD,), jnp.int32) for _ in range(4)],
            *[pltpu.VMEM((L_PAD, 48), jnp.float32) for _ in range(4)],
            *[pltpu.SemaphoreType.DMA for _ in range(6)],
        ],
    )
    return run(*tag_in, *w_in)[:, :, :OUT_D]
